# baseline (speedup 1.0000x reference)
import functools

import jax
import jax.numpy as jnp
from jax import lax
from jax.experimental import pallas as pl
from jax.experimental.pallas import tpu as pltpu
from jax.experimental.pallas import tpu_sc as plsc

NODE_DIM = 128
EDGE_DIM = 16
HID = 272
MT_ROWS = 160


def _lrelu(x, s):
    return jnp.where(x > 0, x, s * x)


def _pick_tile(n, cands):
    for c in cands:
        if n % c == 0:
            return c
    return n


def _sc_gather(node_feats, src, dst):
    N, D = node_feats.shape
    E = src.shape[0]
    NW = 32
    per_w = E // NW
    CH = 80
    n_ch = per_w // CH
    mesh = plsc.VectorSubcoreMesh(core_axis_name="c", subcore_axis_name="s")

    @functools.partial(
        pl.kernel, mesh=mesh,
        out_type=[jax.ShapeDtypeStruct((E, D), jnp.float32),
                  jax.ShapeDtypeStruct((E, D), jnp.float32)],
        scratch_types=[pltpu.VMEM((CH,), jnp.int32),
                       pltpu.VMEM((CH,), jnp.int32),
                       pltpu.VMEM((CH, D), jnp.float32),
                       pltpu.VMEM((CH, D), jnp.float32),
                       pltpu.SemaphoreType.DMA,
                       pltpu.SemaphoreType.DMA],
    )
    def k(table, src_hbm, dst_hbm, hs_out, hd_out, si_v, di_v, sr_v, dr_v,
          s1, s2):
        wid = lax.axis_index("s") * 2 + lax.axis_index("c")
        base = wid * per_w

        def body(i, _):
            off = base + i * CH
            pltpu.sync_copy(src_hbm.at[pl.ds(off, CH)], si_v)
            pltpu.sync_copy(dst_hbm.at[pl.ds(off, CH)], di_v)
            c1 = pltpu.async_copy(table.at[si_v], sr_v, s1)
            c2 = pltpu.async_copy(table.at[di_v], dr_v, s2)
            c1.wait()
            c2.wait()
            pltpu.sync_copy(sr_v, hs_out.at[pl.ds(off, CH)])
            pltpu.sync_copy(dr_v, hd_out.at[pl.ds(off, CH)])
            return 0

        lax.fori_loop(0, n_ch, body, 0)

    return k(node_feats, src, dst)


SEG_CH = 2000


def _edge_stage(h_src, h_dst, edge_feats, p):
    E = h_src.shape[0]
    T = SEG_CH
    grid = (E // T,)
    w1 = p['mb_w1']
    w1s = w1[:NODE_DIM]
    w1d = w1[NODE_DIM:2 * NODE_DIM]
    w1e = w1[2 * NODE_DIM:2 * NODE_DIM + EDGE_DIM]
    wih_t = p['egru_wih'].T
    whh_t = p['egru_whh'].T
    bih = p['egru_bih'].reshape(1, -1)
    bhh = p['egru_bhh'].reshape(1, -1)

    def body(hs_ref, hd_ref, ef_ref, w1s_ref, w1d_ref, w1e_ref, b1_ref,
             g_ref, bb_ref, w2_ref, b2_ref, wih_ref, whh_ref, bih_ref,
             bhh_ref, resw_ref, resb_ref, enew_ref, mt_ref):
        hs = hs_ref[...]
        hd = hd_ref[...]
        ef = ef_ref[...]
        h = (hs @ w1s_ref[...] + hd @ w1d_ref[...] + ef @ w1e_ref[...]
             + b1_ref[...])
        mu = jnp.mean(h, axis=-1, keepdims=True)
        var = jnp.mean((h - mu) ** 2, axis=-1, keepdims=True)
        h = (h - mu) * lax.rsqrt(var + 1e-5) * g_ref[...] + bb_ref[...]
        h = _lrelu(h, 0.2)
        x = h @ w2_ref[...] + b2_ref[...]
        gi = x @ wih_ref[...] + bih_ref[...]
        gh = ef @ whh_ref[...] + bhh_ref[...]
        i_r, i_z, i_n = gi[:, :16], gi[:, 16:32], gi[:, 32:48]
        h_r, h_z, h_n = gh[:, :16], gh[:, 16:32], gh[:, 32:48]
        r = jax.nn.sigmoid(i_r + h_r)
        z = jax.nn.sigmoid(i_z + h_z)
        n = jnp.tanh(i_n + r * h_n)
        e0 = (1.0 - z) * n + z * ef
        e_new = _lrelu(e0 + ef @ resw_ref[...] + resb_ref[...], 0.01)
        enew_ref[...] = e_new
        mt_ref[0, 0:NODE_DIM, :] = hs.T
        mt_ref[0, NODE_DIM:NODE_DIM + EDGE_DIM, :] = e_new.T
        mt_ref[0, NODE_DIM + EDGE_DIM:MT_ROWS, :] = jnp.zeros(
            (MT_ROWS - NODE_DIM - EDGE_DIM, e_new.shape[0]), jnp.float32)

    def rowspec(d):
        return pl.BlockSpec((T, d), lambda i: (i, 0))

    def wspec(a):
        return pl.BlockSpec(a.shape, lambda i: (0,) * a.ndim)

    ws = [w1s, w1d, w1e, p['mb_b1'].reshape(1, -1),
          p['mb_ln_g'].reshape(1, -1), p['mb_ln_b'].reshape(1, -1),
          p['mb_w2'], p['mb_b2'].reshape(1, -1), wih_t, whh_t, bih, bhh,
          p['eres_w'], p['eres_b'].reshape(1, -1)]
    return pl.pallas_call(
        body,
        grid=grid,
        in_specs=[rowspec(NODE_DIM), rowspec(NODE_DIM), rowspec(EDGE_DIM)]
                 + [wspec(a) for a in ws],
        out_specs=[rowspec(EDGE_DIM),
                   pl.BlockSpec((1, MT_ROWS, T), lambda i: (i, 0, 0))],
        out_shape=[jax.ShapeDtypeStruct((E, EDGE_DIM), jnp.float32),
                   jax.ShapeDtypeStruct((E // T, MT_ROWS, T), jnp.float32)],
    )(h_src, h_dst, edge_feats, *ws)


def _sc_segment(m_t, dst, npad):
    E = m_t.shape[0] // MT_ROWS
    NW = 32
    CPW = MT_ROWS // NW
    CH = SEG_CH
    NB = CH // 16
    n_ch = E // CH
    mesh = plsc.VectorSubcoreMesh(core_axis_name="c", subcore_axis_name="s")

    @functools.partial(
        pl.kernel, mesh=mesh,
        out_type=[jax.ShapeDtypeStruct((MT_ROWS * npad,), jnp.float32),
                  jax.ShapeDtypeStruct((MT_ROWS * npad,), jnp.float32)],
        scratch_types=[pltpu.VMEM((CPW * npad,), jnp.float32),
                       pltpu.VMEM((CPW * npad,), jnp.float32),
                       pltpu.VMEM((CPW * CH,), jnp.float32),
                       pltpu.VMEM((CPW * CH,), jnp.float32),
                       pltpu.VMEM((CH,), jnp.int32),
                       pltpu.VMEM((CH,), jnp.int32),
                       pltpu.VMEM((2048,), jnp.int32),
                       pltpu.SemaphoreType.DMA,
                       pltpu.SemaphoreType.DMA,
                       pltpu.SemaphoreType.DMA,
                       pltpu.SemaphoreType.DMA],
        compiler_params=pltpu.CompilerParams(needs_layout_passes=False),
    )
    def k(mt_hbm, dst_hbm, st_out, mx_out, acc_s, acc_m, mbuf0, mbuf1,
          dbuf0, dbuf1, dtmp, sm0, sm1, sd0, sd1):
        wid = lax.axis_index("s") * 2 + lax.axis_index("c")
        r0 = wid * CPW
        zero = jnp.zeros((16,), jnp.float32)
        ninf = jnp.full((16,), -jnp.inf, jnp.float32)
        lid = lax.iota(jnp.int32, 16)
        mbufs = (mbuf0, mbuf1)
        dbufs = (dbuf0, dbuf1)
        sms = (sm0, sm1)
        sds = (sd0, sd1)

        def init_body(i, _):
            acc_s[pl.ds(i * 16, 16)] = zero
            acc_m[pl.ds(i * 16, 16)] = ninf
            return 0

        lax.fori_loop(0, CPW * npad // 16, init_body, 0)

        def issue(c, par):
            pltpu.async_copy(
                mt_hbm.at[pl.ds((c * MT_ROWS + r0) * CH, CPW * CH)],
                mbufs[par], sms[par])
            pltpu.async_copy(dst_hbm.at[pl.ds(c * CH, CH)],
                             dbufs[par], sds[par])

        def process(mbuf, dbuf):
            def batch_body(b, _):
                d = dbuf[pl.ds(b * 16, 16)]
                hw = jnp.bitwise_and(d, 2047)
                plsc.store_scatter(dtmp, [hw], lid)
                got = plsc.load_gather(dtmp, [hw])
                clean = jnp.all(got == lid)
                djs = []
                vals = []
                pends = []
                for j in range(CPW):
                    dj = d + j * npad
                    val = mbuf[pl.ds(j * CH + b * 16, 16)]
                    plsc.addupdate_scatter(acc_s, [dj], val)
                    old = plsc.load_gather(acc_m, [dj])
                    djs.append(dj)
                    vals.append(val)
                    pends.append(val > old)

                @pl.when(clean)
                def _():
                    for j in range(CPW):
                        plsc.store_scatter(acc_m, [djs[j]], vals[j],
                                           mask=pends[j])

                @pl.when(jnp.logical_not(clean))
                def _():
                    for j in range(CPW):
                        dj, val, pend = djs[j], vals[j], pends[j]
                        for _r in range(4):
                            plsc.store_scatter(acc_m, [dj], val, mask=pend)
                            cur = plsc.load_gather(acc_m, [dj])
                            pend = cur < val
                return 0

            lax.fori_loop(0, NB, batch_body, 0)

        def drain(par):
            pltpu.make_async_copy(
                mt_hbm.at[pl.ds(r0 * CH, CPW * CH)], mbufs[par],
                sms[par]).wait()
            pltpu.make_async_copy(
                dst_hbm.at[pl.ds(0, CH)], dbufs[par], sds[par]).wait()

        issue(0, 0)

        def pair_body(c2, _):
            for par in (0, 1):
                c = 2 * c2 + par

                @pl.when(c + 1 < n_ch)
                def _():
                    issue(c + 1, 1 - par)

                drain(par)
                process(mbufs[par], dbufs[par])
            return 0

        lax.fori_loop(0, n_ch // 2, pair_body, 0)
        for j in range(CPW):
            pltpu.sync_copy(acc_s.at[pl.ds(j * npad, npad)],
                            st_out.at[pl.ds((r0 + j) * npad, npad)])
            pltpu.sync_copy(acc_m.at[pl.ds(j * npad, npad)],
                            mx_out.at[pl.ds((r0 + j) * npad, npad)])

    s_t, mx_t = k(m_t, dst)
    return (s_t.reshape(MT_ROWS, npad), mx_t.reshape(MT_ROWS, npad))


def _node_stage(s_t, mx_t, node_feats, p):
    N = node_feats.shape[0]
    T = _pick_tile(N, (2048, 1024, 512, 256, 128))
    grid = (N // T,)
    D = NODE_DIM + EDGE_DIM
    w1 = p['mb_w1']
    pad = jnp.zeros((MT_ROWS - D, HID), jnp.float32)
    w1s = jnp.concatenate([w1[:D], pad], axis=0)
    w1m = jnp.concatenate([w1[D:2 * D], pad], axis=0)
    wih_t = p['agru_wih'].T
    whh_t = p['agru_whh'].T
    bih = p['agru_bih'].reshape(1, -1)
    bhh = p['agru_bhh'].reshape(1, -1)

    def body(st_ref, mt_ref, nf_ref, w1s_ref, w1m_ref, b1_ref, g_ref, bb_ref,
             w2_ref, b2_ref, wih_ref, whh_ref, bih_ref, bhh_ref, resw_ref,
             resb_ref, hnew_ref):
        st = st_ref[...]
        mt = mt_ref[...]
        mt = jnp.where(jnp.isfinite(mt), mt, 0.0)
        nf = nf_ref[...]
        dn = (((0,), (0,)), ((), ()))
        h = (lax.dot_general(st, w1s_ref[...], dn,
                             preferred_element_type=jnp.float32)
             + lax.dot_general(mt, w1m_ref[...], dn,
                               preferred_element_type=jnp.float32)
             + b1_ref[...])
        mu = jnp.mean(h, axis=-1, keepdims=True)
        var = jnp.mean((h - mu) ** 2, axis=-1, keepdims=True)
        h = (h - mu) * lax.rsqrt(var + 1e-5) * g_ref[...] + bb_ref[...]
        h = _lrelu(h, 0.2)
        x = h @ w2_ref[...] + b2_ref[...]
        gi = x @ wih_ref[...] + bih_ref[...]
        gh = nf @ whh_ref[...] + bhh_ref[...]
        K = NODE_DIM
        i_r, i_z, i_n = gi[:, :K], gi[:, K:2 * K], gi[:, 2 * K:3 * K]
        h_r, h_z, h_n = gh[:, :K], gh[:, K:2 * K], gh[:, 2 * K:3 * K]
        r = jax.nn.sigmoid(i_r + h_r)
        z = jax.nn.sigmoid(i_z + h_z)
        n = jnp.tanh(i_n + r * h_n)
        h0 = (1.0 - z) * n + z * nf
        h_new = _lrelu(h0 + nf @ resw_ref[...] + resb_ref[...], 0.01)
        hnew_ref[...] = h_new

    def wspec(a):
        return pl.BlockSpec(a.shape, lambda i: (0,) * a.ndim)

    ws = [w1s, w1m, p['mb_b1'].reshape(1, -1), p['mb_ln_g'].reshape(1, -1),
          p['mb_ln_b'].reshape(1, -1), p['mb_w2'],
          p['mb_b2'].reshape(1, -1), wih_t, whh_t, bih, bhh,
          p['ares_w'], p['ares_b'].reshape(1, -1)]
    return pl.pallas_call(
        body,
        grid=grid,
        in_specs=[pl.BlockSpec((MT_ROWS, T), lambda i: (0, i)),
                  pl.BlockSpec((MT_ROWS, T), lambda i: (0, i)),
                  pl.BlockSpec((T, NODE_DIM), lambda i: (i, 0))]
                 + [wspec(a) for a in ws],
        out_specs=pl.BlockSpec((T, NODE_DIM), lambda i: (i, 0)),
        out_shape=jax.ShapeDtypeStruct((N, NODE_DIM), jnp.float32),
    )(s_t, mx_t, node_feats, *ws)


def kernel(node_feats, edge_feats, params, edge_index):
    n = node_feats.shape[0]
    src = edge_index[0]
    dst = edge_index[1]
    h_src, h_dst = _sc_gather(node_feats, src, dst)
    e_new, m_t = _edge_stage(h_src, h_dst, edge_feats, params)
    npad = ((n + 2047) // 2048) * 2048
    s_t, mx_t = _sc_segment(m_t.reshape(-1), dst, npad)
    nf_pad = jnp.pad(node_feats, ((0, npad - n), (0, 0)))
    h_new = _node_stage(s_t, mx_t, nf_pad, params)[:n]
    return (h_new, e_new)


# speedup vs baseline: 1.2647x; 1.2647x over previous
import functools

import jax
import jax.numpy as jnp
from jax import lax
from jax.experimental import pallas as pl
from jax.experimental.pallas import tpu as pltpu
from jax.experimental.pallas import tpu_sc as plsc

NODE_DIM = 128
EDGE_DIM = 16
HID = 272
MT_ROWS = 160


def _lrelu(x, s):
    return jnp.where(x > 0, x, s * x)


def _pick_tile(n, cands):
    for c in cands:
        if n % c == 0:
            return c
    return n


def _sc_gather(node_feats, src, dst):
    N, D = node_feats.shape
    E = src.shape[0]
    NW = 32
    per_w = E // NW
    CH = 80
    n_ch = per_w // CH
    mesh = plsc.VectorSubcoreMesh(core_axis_name="c", subcore_axis_name="s")

    @functools.partial(
        pl.kernel, mesh=mesh,
        out_type=[jax.ShapeDtypeStruct((E, D), jnp.float32),
                  jax.ShapeDtypeStruct((E, D), jnp.float32)],
        scratch_types=[pltpu.VMEM((CH,), jnp.int32),
                       pltpu.VMEM((CH,), jnp.int32),
                       pltpu.VMEM((CH, D), jnp.float32),
                       pltpu.VMEM((CH, D), jnp.float32),
                       pltpu.SemaphoreType.DMA,
                       pltpu.SemaphoreType.DMA],
    )
    def k(table, src_hbm, dst_hbm, hs_out, hd_out, si_v, di_v, sr_v, dr_v,
          s1, s2):
        wid = lax.axis_index("s") * 2 + lax.axis_index("c")
        base = wid * per_w

        def body(i, _):
            off = base + i * CH
            pltpu.sync_copy(src_hbm.at[pl.ds(off, CH)], si_v)
            pltpu.sync_copy(dst_hbm.at[pl.ds(off, CH)], di_v)
            c1 = pltpu.async_copy(table.at[si_v], sr_v, s1)
            c2 = pltpu.async_copy(table.at[di_v], dr_v, s2)
            c1.wait()
            c2.wait()
            pltpu.sync_copy(sr_v, hs_out.at[pl.ds(off, CH)])
            pltpu.sync_copy(dr_v, hd_out.at[pl.ds(off, CH)])
            return 0

        lax.fori_loop(0, n_ch, body, 0)

    return k(node_feats, src, dst)


SEG_CH = 2000


def _edge_stage(h_src, h_dst, edge_feats, p):
    E = h_src.shape[0]
    T = SEG_CH
    grid = (E // T,)
    w1 = p['mb_w1']
    w1s = w1[:NODE_DIM]
    w1d = w1[NODE_DIM:2 * NODE_DIM]
    w1e = w1[2 * NODE_DIM:2 * NODE_DIM + EDGE_DIM]
    wih_t = p['egru_wih'].T
    whh_t = p['egru_whh'].T
    bih = p['egru_bih'].reshape(1, -1)
    bhh = p['egru_bhh'].reshape(1, -1)

    def body(hs_ref, hd_ref, ef_ref, w1s_ref, w1d_ref, w1e_ref, b1_ref,
             g_ref, bb_ref, w2_ref, b2_ref, wih_ref, whh_ref, bih_ref,
             bhh_ref, resw_ref, resb_ref, enew_ref, mt_ref):
        hs = hs_ref[...]
        hd = hd_ref[...]
        ef = ef_ref[...]
        h = (hs @ w1s_ref[...] + hd @ w1d_ref[...] + ef @ w1e_ref[...]
             + b1_ref[...])
        mu = jnp.mean(h, axis=-1, keepdims=True)
        var = jnp.mean((h - mu) ** 2, axis=-1, keepdims=True)
        h = (h - mu) * lax.rsqrt(var + 1e-5) * g_ref[...] + bb_ref[...]
        h = _lrelu(h, 0.2)
        x = h @ w2_ref[...] + b2_ref[...]
        gi = x @ wih_ref[...] + bih_ref[...]
        gh = ef @ whh_ref[...] + bhh_ref[...]
        i_r, i_z, i_n = gi[:, :16], gi[:, 16:32], gi[:, 32:48]
        h_r, h_z, h_n = gh[:, :16], gh[:, 16:32], gh[:, 32:48]
        r = jax.nn.sigmoid(i_r + h_r)
        z = jax.nn.sigmoid(i_z + h_z)
        n = jnp.tanh(i_n + r * h_n)
        e0 = (1.0 - z) * n + z * ef
        e_new = _lrelu(e0 + ef @ resw_ref[...] + resb_ref[...], 0.01)
        enew_ref[...] = e_new
        mt_ref[0, 0:NODE_DIM, :] = hs.T
        mt_ref[0, NODE_DIM:NODE_DIM + EDGE_DIM, :] = e_new.T
        mt_ref[0, NODE_DIM + EDGE_DIM:MT_ROWS, :] = jnp.zeros(
            (MT_ROWS - NODE_DIM - EDGE_DIM, e_new.shape[0]), jnp.float32)

    def rowspec(d):
        return pl.BlockSpec((T, d), lambda i: (i, 0))

    def wspec(a):
        return pl.BlockSpec(a.shape, lambda i: (0,) * a.ndim)

    ws = [w1s, w1d, w1e, p['mb_b1'].reshape(1, -1),
          p['mb_ln_g'].reshape(1, -1), p['mb_ln_b'].reshape(1, -1),
          p['mb_w2'], p['mb_b2'].reshape(1, -1), wih_t, whh_t, bih, bhh,
          p['eres_w'], p['eres_b'].reshape(1, -1)]
    return pl.pallas_call(
        body,
        grid=grid,
        in_specs=[rowspec(NODE_DIM), rowspec(NODE_DIM), rowspec(EDGE_DIM)]
                 + [wspec(a) for a in ws],
        out_specs=[rowspec(EDGE_DIM),
                   pl.BlockSpec((1, MT_ROWS, T), lambda i: (i, 0, 0))],
        out_shape=[jax.ShapeDtypeStruct((E, EDGE_DIM), jnp.float32),
                   jax.ShapeDtypeStruct((E // T, MT_ROWS, T), jnp.float32)],
    )(h_src, h_dst, edge_feats, *ws)


def _sc_segment(m_t, dst, npad):
    E = m_t.shape[0] // MT_ROWS
    NW = 32
    CPW = MT_ROWS // NW
    CH = SEG_CH
    NB = CH // 16
    n_ch = E // CH
    mesh = plsc.VectorSubcoreMesh(core_axis_name="c", subcore_axis_name="s")

    @functools.partial(
        pl.kernel, mesh=mesh,
        out_type=[jax.ShapeDtypeStruct((MT_ROWS * npad,), jnp.float32),
                  jax.ShapeDtypeStruct((MT_ROWS * npad,), jnp.float32)],
        scratch_types=[pltpu.VMEM((CPW * npad,), jnp.float32),
                       pltpu.VMEM((CPW * npad,), jnp.float32),
                       pltpu.VMEM((CPW * CH,), jnp.float32),
                       pltpu.VMEM((CPW * CH,), jnp.float32),
                       pltpu.VMEM((CH,), jnp.int32),
                       pltpu.VMEM((CH,), jnp.int32),
                       pltpu.VMEM((2048,), jnp.int32),
                       pltpu.SemaphoreType.DMA,
                       pltpu.SemaphoreType.DMA,
                       pltpu.SemaphoreType.DMA,
                       pltpu.SemaphoreType.DMA],
        compiler_params=pltpu.CompilerParams(needs_layout_passes=False),
    )
    def k(mt_hbm, dst_hbm, st_out, mx_out, acc_s, acc_m, mbuf0, mbuf1,
          dbuf0, dbuf1, dtmp, sm0, sm1, sd0, sd1):
        wid = lax.axis_index("s") * 2 + lax.axis_index("c")
        r0 = wid * CPW
        zero = jnp.zeros((16,), jnp.float32)
        ninf = jnp.full((16,), -jnp.inf, jnp.float32)
        lid = lax.iota(jnp.int32, 16)
        mbufs = (mbuf0, mbuf1)
        dbufs = (dbuf0, dbuf1)
        sms = (sm0, sm1)
        sds = (sd0, sd1)

        def init_body(i, _):
            acc_s[pl.ds(i * 16, 16)] = zero
            acc_m[pl.ds(i * 16, 16)] = ninf
            return 0

        lax.fori_loop(0, CPW * npad // 16, init_body, 0)

        def issue(c, par):
            pltpu.async_copy(
                mt_hbm.at[pl.ds((c * MT_ROWS + r0) * CH, CPW * CH)],
                mbufs[par], sms[par])
            pltpu.async_copy(dst_hbm.at[pl.ds(c * CH, CH)],
                             dbufs[par], sds[par])

        def process(mbuf, dbuf):
            def batch_body(b, _):
                d = dbuf[pl.ds(b * 16, 16)]
                hw = jnp.bitwise_and(d, 2047)
                plsc.store_scatter(dtmp, [hw], lid)
                got = plsc.load_gather(dtmp, [hw])
                clean = jnp.all(got == lid)

                @pl.when(clean)
                def _():
                    for j in range(CPW):
                        dj = d + j * npad
                        val = mbuf[pl.ds(j * CH + b * 16, 16)]
                        plsc.addupdate_scatter(acc_s, [dj], val)
                        old = plsc.load_gather(acc_m, [dj])
                        plsc.store_scatter(acc_m, [dj], val, mask=val > old)

                @pl.when(jnp.logical_not(clean))
                def _():
                    for j in range(CPW):
                        dj = d + j * npad
                        val = mbuf[pl.ds(j * CH + b * 16, 16)]
                        plsc.addupdate_scatter(acc_s, [dj], val)
                        old = plsc.load_gather(acc_m, [dj])
                        pend = val > old
                        for _r in range(4):
                            plsc.store_scatter(acc_m, [dj], val, mask=pend)
                            cur = plsc.load_gather(acc_m, [dj])
                            pend = cur < val
                return 0

            lax.fori_loop(0, NB, batch_body, 0)

        def drain(par):
            pltpu.make_async_copy(
                mt_hbm.at[pl.ds(r0 * CH, CPW * CH)], mbufs[par],
                sms[par]).wait()
            pltpu.make_async_copy(
                dst_hbm.at[pl.ds(0, CH)], dbufs[par], sds[par]).wait()

        issue(0, 0)

        def pair_body(c2, _):
            for par in (0, 1):
                c = 2 * c2 + par

                @pl.when(c + 1 < n_ch)
                def _():
                    issue(c + 1, 1 - par)

                drain(par)
                process(mbufs[par], dbufs[par])
            return 0

        lax.fori_loop(0, n_ch // 2, pair_body, 0)
        for j in range(CPW):
            pltpu.sync_copy(acc_s.at[pl.ds(j * npad, npad)],
                            st_out.at[pl.ds((r0 + j) * npad, npad)])
            pltpu.sync_copy(acc_m.at[pl.ds(j * npad, npad)],
                            mx_out.at[pl.ds((r0 + j) * npad, npad)])

    s_t, mx_t = k(m_t, dst)
    return (s_t.reshape(MT_ROWS, npad), mx_t.reshape(MT_ROWS, npad))


def _node_stage(s_t, mx_t, node_feats, p):
    N = node_feats.shape[0]
    T = _pick_tile(N, (2048, 1024, 512, 256, 128))
    grid = (N // T,)
    D = NODE_DIM + EDGE_DIM
    w1 = p['mb_w1']
    pad = jnp.zeros((MT_ROWS - D, HID), jnp.float32)
    w1s = jnp.concatenate([w1[:D], pad], axis=0)
    w1m = jnp.concatenate([w1[D:2 * D], pad], axis=0)
    wih_t = p['agru_wih'].T
    whh_t = p['agru_whh'].T
    bih = p['agru_bih'].reshape(1, -1)
    bhh = p['agru_bhh'].reshape(1, -1)

    def body(st_ref, mt_ref, nf_ref, w1s_ref, w1m_ref, b1_ref, g_ref, bb_ref,
             w2_ref, b2_ref, wih_ref, whh_ref, bih_ref, bhh_ref, resw_ref,
             resb_ref, hnew_ref):
        st = st_ref[...]
        mt = mt_ref[...]
        mt = jnp.where(jnp.isfinite(mt), mt, 0.0)
        nf = nf_ref[...]
        dn = (((0,), (0,)), ((), ()))
        h = (lax.dot_general(st, w1s_ref[...], dn,
                             preferred_element_type=jnp.float32)
             + lax.dot_general(mt, w1m_ref[...], dn,
                               preferred_element_type=jnp.float32)
             + b1_ref[...])
        mu = jnp.mean(h, axis=-1, keepdims=True)
        var = jnp.mean((h - mu) ** 2, axis=-1, keepdims=True)
        h = (h - mu) * lax.rsqrt(var + 1e-5) * g_ref[...] + bb_ref[...]
        h = _lrelu(h, 0.2)
        x = h @ w2_ref[...] + b2_ref[...]
        gi = x @ wih_ref[...] + bih_ref[...]
        gh = nf @ whh_ref[...] + bhh_ref[...]
        K = NODE_DIM
        i_r, i_z, i_n = gi[:, :K], gi[:, K:2 * K], gi[:, 2 * K:3 * K]
        h_r, h_z, h_n = gh[:, :K], gh[:, K:2 * K], gh[:, 2 * K:3 * K]
        r = jax.nn.sigmoid(i_r + h_r)
        z = jax.nn.sigmoid(i_z + h_z)
        n = jnp.tanh(i_n + r * h_n)
        h0 = (1.0 - z) * n + z * nf
        h_new = _lrelu(h0 + nf @ resw_ref[...] + resb_ref[...], 0.01)
        hnew_ref[...] = h_new

    def wspec(a):
        return pl.BlockSpec(a.shape, lambda i: (0,) * a.ndim)

    ws = [w1s, w1m, p['mb_b1'].reshape(1, -1), p['mb_ln_g'].reshape(1, -1),
          p['mb_ln_b'].reshape(1, -1), p['mb_w2'],
          p['mb_b2'].reshape(1, -1), wih_t, whh_t, bih, bhh,
          p['ares_w'], p['ares_b'].reshape(1, -1)]
    return pl.pallas_call(
        body,
        grid=grid,
        in_specs=[pl.BlockSpec((MT_ROWS, T), lambda i: (0, i)),
                  pl.BlockSpec((MT_ROWS, T), lambda i: (0, i)),
                  pl.BlockSpec((T, NODE_DIM), lambda i: (i, 0))]
                 + [wspec(a) for a in ws],
        out_specs=pl.BlockSpec((T, NODE_DIM), lambda i: (i, 0)),
        out_shape=jax.ShapeDtypeStruct((N, NODE_DIM), jnp.float32),
    )(s_t, mx_t, node_feats, *ws)


def kernel(node_feats, edge_feats, params, edge_index):
    n = node_feats.shape[0]
    src = edge_index[0]
    dst = edge_index[1]
    h_src, h_dst = _sc_gather(node_feats, src, dst)
    e_new, m_t = _edge_stage(h_src, h_dst, edge_feats, params)
    npad = ((n + 2047) // 2048) * 2048
    s_t, mx_t = _sc_segment(m_t.reshape(-1), dst, npad)
    nf_pad = jnp.pad(node_feats, ((0, npad - n), (0, 0)))
    h_new = _node_stage(s_t, mx_t, nf_pad, params)[:n]
    return (h_new, e_new)
